# baseline (speedup 1.0000x reference)
import functools

import jax
import jax.numpy as jnp
from jax import lax
from jax.experimental import pallas as pl
from jax.experimental.pallas import tpu as pltpu
from jax.experimental.pallas import tpu_sc as plsc

N_NODES = 10000
N_EDGES = 10000
N_INC = 320000
D = 128
DP = 144

NC = 2
NS = 16
NW = NC * NS
K = 64
CH = 158
INC_PAD = NW * CH * K
E_PAD = 10112
ROWS_PER_TILE = E_PAD // NS

_mesh = plsc.VectorSubcoreMesh(
    core_axis_name="c", subcore_axis_name="s", num_cores=NC, num_subcores=NS)


@functools.partial(
    pl.kernel,
    out_type=jax.ShapeDtypeStruct((NC, E_PAD, DP), jnp.float32),
    mesh=_mesh,
    scratch_types=[
        pltpu.VMEM((CH, K), jnp.int32),
        pltpu.VMEM((CH, K), jnp.int32),
        pltpu.VMEM((K, DP), jnp.float32),
        pltpu.VMEM((K, DP), jnp.float32),
        pltpu.VMEM_SHARED((E_PAD, DP), jnp.float32),
        pltpu.SemaphoreType.DMA,
        pltpu.SemaphoreType.DMA,
    ],
    compiler_params=pltpu.CompilerParams(use_tc_tiling_on_sc=False),
)
def _sc_gather_scatter_add(src_hbm, gidx_hbm, sidx_hbm, zeros_hbm, out_hbm,
                           gidx_v, sidx_v, rows0_v, rows1_v, acc_sh,
                           sem0, sem1):
    cid = lax.axis_index("c")
    sid = lax.axis_index("s")
    wid = cid * NS + sid
    rows = (rows0_v, rows1_v)
    sems = (sem0, sem1)

    pltpu.sync_copy(zeros_hbm, acc_sh.at[pl.ds(sid * ROWS_PER_TILE, ROWS_PER_TILE)])

    pltpu.sync_copy(gidx_hbm.at[pl.ds(wid * CH, CH)], gidx_v)
    pltpu.sync_copy(sidx_hbm.at[pl.ds(wid * CH, CH)], sidx_v)
    plsc.subcore_barrier()

    pltpu.async_copy(src_hbm.at[gidx_v.at[0]], rows[0], sems[0])
    pltpu.async_copy(src_hbm.at[gidx_v.at[1]], rows[1], sems[1])

    @pl.loop(0, CH, step=2)
    def _chunk(j):
        for t in range(2):
            g = j + t
            pltpu.make_async_copy(src_hbm.at[gidx_v.at[g]], rows[t], sems[t]).wait()
            pltpu.sync_copy(rows[t], acc_sh.at[sidx_v.at[g]], add=True)

            @pl.when(g + 2 < CH)
            def _():
                pltpu.async_copy(src_hbm.at[gidx_v.at[g + 2]], rows[t], sems[t])

    plsc.subcore_barrier()
    pltpu.sync_copy(acc_sh.at[pl.ds(sid * ROWS_PER_TILE, ROWS_PER_TILE)],
                    out_hbm.at[cid, pl.ds(sid * ROWS_PER_TILE, ROWS_PER_TILE)])


_R = 1000


def _combine_div_body(acc_ref, out_ref):
    s = acc_ref[0] + acc_ref[1]
    cnt = s[:, D:D + 1]
    ef = s[:, :D] / jnp.maximum(cnt, 1.0)
    col = lax.broadcasted_iota(jnp.int32, (_R, DP - D), 1)
    tail = jnp.where(col == 0, 1.0, 0.0).astype(jnp.float32)
    out_ref[...] = jnp.concatenate([ef, tail], axis=1)


def _finish_body(acc_ref, w_ref, b_ref, out_ref):
    s = acc_ref[0] + acc_ref[1]
    y = lax.dot_general(s[:, :D], w_ref[...], (((1,), (1,)), ((), ())),
                        preferred_element_type=jnp.float32)
    out_ref[...] = y + s[:, D:D + 1] * b_ref[...]


def kernel(x, hyperedge_index, W, b):
    ones = jnp.ones((N_NODES, 1), jnp.float32)
    pad = jnp.zeros((N_NODES, DP - D - 1), jnp.float32)
    x_pad = jnp.concatenate([x, ones, pad], axis=1)

    npad = INC_PAD - N_INC
    gpad = jnp.zeros((npad,), jnp.int32)
    spad = jnp.full((npad,), N_EDGES, jnp.int32)
    nidx_g = jnp.concatenate([hyperedge_index[0], gpad]).reshape(NW * CH, K)
    nidx_s = jnp.concatenate([hyperedge_index[0], spad]).reshape(NW * CH, K)
    eidx_g = jnp.concatenate([hyperedge_index[1], gpad]).reshape(NW * CH, K)
    eidx_s = jnp.concatenate([hyperedge_index[1], spad]).reshape(NW * CH, K)
    zeros = jnp.zeros((ROWS_PER_TILE, DP), jnp.float32)

    part_a = _sc_gather_scatter_add(x_pad, nidx_g, eidx_s, zeros)

    ef_pad = pl.pallas_call(
        _combine_div_body,
        grid=(N_EDGES // _R,),
        in_specs=[pl.BlockSpec((NC, _R, DP), lambda i: (0, i, 0))],
        out_specs=pl.BlockSpec((_R, DP), lambda i: (i, 0)),
        out_shape=jax.ShapeDtypeStruct((N_EDGES, DP), jnp.float32),
    )(part_a)

    part_b = _sc_gather_scatter_add(ef_pad, eidx_g, nidx_s, zeros)

    out = pl.pallas_call(
        _finish_body,
        grid=(N_NODES // _R,),
        in_specs=[
            pl.BlockSpec((NC, _R, DP), lambda i: (0, i, 0)),
            pl.BlockSpec((D, D), lambda i: (0, 0)),
            pl.BlockSpec((1, D), lambda i: (0, 0)),
        ],
        out_specs=pl.BlockSpec((_R, D), lambda i: (i, 0)),
        out_shape=jax.ShapeDtypeStruct((N_NODES, D), jnp.float32),
    )(part_b, W, b.reshape(1, D))
    return out


# speedup vs baseline: 1.4340x; 1.4340x over previous
import functools

import jax
import jax.numpy as jnp
from jax import lax
from jax.experimental import pallas as pl
from jax.experimental.pallas import tpu as pltpu
from jax.experimental.pallas import tpu_sc as plsc

N_NODES = 10000
N_EDGES = 10000
N_INC = 320000
D = 128
DP = 144

NC = 2
NS = 16
NW = NC * NS
K = 125
CH = 80
INC_PAD = NW * CH * K
E_PAD = 10112
ROWS_PER_TILE = E_PAD // NS

_mesh = plsc.VectorSubcoreMesh(
    core_axis_name="c", subcore_axis_name="s", num_cores=NC, num_subcores=NS)


@functools.partial(
    pl.kernel,
    out_type=jax.ShapeDtypeStruct((NC, E_PAD, DP), jnp.float32),
    mesh=_mesh,
    scratch_types=[
        pltpu.VMEM((CH, K), jnp.int32),
        pltpu.VMEM((CH, K), jnp.int32),
        pltpu.VMEM((K, DP), jnp.float32),
        pltpu.VMEM_SHARED((E_PAD, DP), jnp.float32),
    ],
    compiler_params=pltpu.CompilerParams(use_tc_tiling_on_sc=False),
)
def _sc_gather_scatter_add(src_hbm, gidx_hbm, sidx_hbm, zeros_hbm, out_hbm,
                           gidx_v, sidx_v, rows_v, acc_sh):
    cid = lax.axis_index("c")
    sid = lax.axis_index("s")
    wid = cid * NS + sid

    pltpu.sync_copy(zeros_hbm, acc_sh.at[pl.ds(sid * ROWS_PER_TILE, ROWS_PER_TILE)])

    pltpu.sync_copy(gidx_hbm.at[pl.ds(wid * CH, CH)], gidx_v)
    pltpu.sync_copy(sidx_hbm.at[pl.ds(wid * CH, CH)], sidx_v)
    plsc.subcore_barrier()

    @pl.loop(0, CH)
    def _chunk(j):
        pltpu.sync_copy(src_hbm.at[gidx_v.at[j]], rows_v)
        pltpu.sync_copy(rows_v, acc_sh.at[sidx_v.at[j]], add=True)

    plsc.subcore_barrier()
    pltpu.sync_copy(acc_sh.at[pl.ds(sid * ROWS_PER_TILE, ROWS_PER_TILE)],
                    out_hbm.at[cid, pl.ds(sid * ROWS_PER_TILE, ROWS_PER_TILE)])


_R = 1000


def _combine_div_body(acc_ref, out_ref):
    s = acc_ref[0] + acc_ref[1]
    cnt = s[:, D:D + 1]
    ef = s[:, :D] / jnp.maximum(cnt, 1.0)
    col = lax.broadcasted_iota(jnp.int32, (_R, DP - D), 1)
    tail = jnp.where(col == 0, 1.0, 0.0).astype(jnp.float32)
    out_ref[...] = jnp.concatenate([ef, tail], axis=1)


def _finish_body(acc_ref, w_ref, b_ref, out_ref):
    s = acc_ref[0] + acc_ref[1]
    y = lax.dot_general(s[:, :D], w_ref[...], (((1,), (1,)), ((), ())),
                        preferred_element_type=jnp.float32)
    out_ref[...] = y + s[:, D:D + 1] * b_ref[...]


def kernel(x, hyperedge_index, W, b):
    ones = jnp.ones((N_NODES, 1), jnp.float32)
    pad = jnp.zeros((N_NODES, DP - D - 1), jnp.float32)
    x_pad = jnp.concatenate([x, ones, pad], axis=1)

    npad = INC_PAD - N_INC
    gpad = jnp.zeros((npad,), jnp.int32)
    spad = jnp.full((npad,), N_EDGES, jnp.int32)
    nidx_g = jnp.concatenate([hyperedge_index[0], gpad]).reshape(NW * CH, K)
    nidx_s = jnp.concatenate([hyperedge_index[0], spad]).reshape(NW * CH, K)
    eidx_g = jnp.concatenate([hyperedge_index[1], gpad]).reshape(NW * CH, K)
    eidx_s = jnp.concatenate([hyperedge_index[1], spad]).reshape(NW * CH, K)
    zeros = jnp.zeros((ROWS_PER_TILE, DP), jnp.float32)

    part_a = _sc_gather_scatter_add(x_pad, nidx_g, eidx_s, zeros)

    ef_pad = pl.pallas_call(
        _combine_div_body,
        grid=(N_EDGES // _R,),
        in_specs=[pl.BlockSpec((NC, _R, DP), lambda i: (0, i, 0))],
        out_specs=pl.BlockSpec((_R, DP), lambda i: (i, 0)),
        out_shape=jax.ShapeDtypeStruct((N_EDGES, DP), jnp.float32),
    )(part_a)

    part_b = _sc_gather_scatter_add(ef_pad, eidx_g, nidx_s, zeros)

    out = pl.pallas_call(
        _finish_body,
        grid=(N_NODES // _R,),
        in_specs=[
            pl.BlockSpec((NC, _R, DP), lambda i: (0, i, 0)),
            pl.BlockSpec((D, D), lambda i: (0, 0)),
            pl.BlockSpec((1, D), lambda i: (0, 0)),
        ],
        out_specs=pl.BlockSpec((_R, D), lambda i: (i, 0)),
        out_shape=jax.ShapeDtypeStruct((N_NODES, D), jnp.float32),
    )(part_b, W, b.reshape(1, D))
    return out
